# Initial kernel scaffold; baseline (speedup 1.0000x reference)
import jax
import jax.numpy as jnp
from jax.experimental import pallas as pl


def kernel(node_embeddings, node_depths, max_depth):
    raise NotImplementedError("write your pallas kernel here")


# speedup vs baseline: 9.7706x; 9.7706x over previous
import functools

import jax
import jax.numpy as jnp
from jax import lax
from jax.experimental import pallas as pl
from jax.experimental.pallas import tpu as pltpu

NUM_SEG = 33
BR = 2000
NB = 50


def _pool_body(depths_ref, emb_ref, out_ref, sum_s, max_s, cnt_s, *, num_blocks):
    i = pl.program_id(0)

    @pl.when(i == 0)
    def _init():
        sum_s[...] = jnp.zeros_like(sum_s)
        cnt_s[...] = jnp.zeros_like(cnt_s)
        max_s[...] = jnp.full_like(max_s, -jnp.inf)

    d = depths_ref[0, 0, :]
    emb = emb_ref[...]

    seg_ids = jax.lax.broadcasted_iota(jnp.int32, (BR, NUM_SEG), 1)
    oh = (d[:, None] == seg_ids).astype(jnp.float32)

    dims = (((0,), (0,)), ((), ()))
    sum_s[...] += lax.dot_general(oh, emb, dims, preferred_element_type=jnp.float32)
    cnt_s[...] += jnp.sum(oh, axis=0)[:, None]

    dmin = jnp.min(d)
    dmax = jnp.max(d)
    for s in range(NUM_SEG):
        @pl.when((dmin <= s) & (s <= dmax))
        def _seg_max():
            mask = jnp.where(d == s, 0.0, -jnp.inf)[:, None]
            blk = jnp.max(emb + mask, axis=0)
            max_s[s, :] = jnp.maximum(max_s[s, :], blk)

    @pl.when(i == num_blocks - 1)
    def _finish():
        cnt = cnt_s[...]
        mean = sum_s[...] / jnp.maximum(cnt, 1.0)
        nonempty = cnt > 0.0
        out_ref[:, :128] = jnp.where(nonempty, mean, 0.0)
        out_ref[:, 128:] = jnp.where(nonempty, max_s[...], 0.0)


def kernel(node_embeddings, node_depths, max_depth):
    n, f = node_embeddings.shape
    depths3 = jnp.minimum(node_depths, max_depth).astype(jnp.int32).reshape(NB, 1, BR)
    body = functools.partial(_pool_body, num_blocks=NB)
    out = pl.pallas_call(
        body,
        grid=(NB,),
        in_specs=[
            pl.BlockSpec((1, 1, BR), lambda i: (i, 0, 0)),
            pl.BlockSpec((BR, f), lambda i: (i, 0)),
        ],
        out_specs=pl.BlockSpec((NUM_SEG, 2 * f), lambda i: (0, 0)),
        out_shape=jax.ShapeDtypeStruct((NUM_SEG, 2 * f), jnp.float32),
        scratch_shapes=[
            pltpu.VMEM((NUM_SEG, f), jnp.float32),
            pltpu.VMEM((NUM_SEG, f), jnp.float32),
            pltpu.VMEM((NUM_SEG, 1), jnp.float32),
        ],
    )(depths3, node_embeddings)
    return out
